# Initial kernel scaffold; baseline (speedup 1.0000x reference)
import jax
import jax.numpy as jnp
from jax.experimental import pallas as pl


def kernel(z, edge_index, W1, b1, W2, b2, Wfc, bfc):
    raise NotImplementedError("write your pallas kernel here")


# speedup vs baseline: 12.8323x; 12.8323x over previous
import functools

import jax
import jax.numpy as jnp
from jax import lax
from jax.experimental import pallas as pl
from jax.experimental.pallas import tpu as pltpu
from jax.experimental.pallas import tpu_sc as plsc

N_NODES = 10000
N_EDGES = 320000
IN_CH = 128
HID = 128
OUT = 64
FC = 1024

NC = 2
NS = 16
NW = NC * NS
E_PER_W = N_EDGES // NW
CHUNK = 80
N_CHUNKS = E_PER_W // CHUNK
RA = 624
TAIL = N_NODES - (NS - 1) * RA - RA
ZR = 104
DEGW = 16

BR = 1000
NB = N_NODES // BR


def _sc_mesh():
    return plsc.VectorSubcoreMesh(core_axis_name="c", subcore_axis_name="s")


def _deg_call(dst):
    @functools.partial(
        pl.kernel,
        out_type=jax.ShapeDtypeStruct((NC * N_NODES,), jnp.float32),
        mesh=_sc_mesh(),
        scratch_types=[
            pltpu.VMEM((CHUNK,), jnp.int32),
            pltpu.VMEM((CHUNK,), jnp.float32),
            pltpu.VMEM((RA,), jnp.float32),
            pltpu.VMEM_SHARED((N_NODES,), jnp.float32),
        ],
    )
    def deg_kernel(dst_hbm, out_hbm, didx, ones_v, zbuf, acc):
        c = lax.axis_index("c")
        s = lax.axis_index("s")
        wid = s * NC + c
        one16 = jnp.ones((16,), jnp.float32)
        zero16 = jnp.zeros((16,), jnp.float32)

        def fill(i, _):
            ones_v[pl.ds(i * 16, 16)] = one16
            return _

        lax.fori_loop(0, CHUNK // 16, fill, 0)

        def fillz(i, _):
            zbuf[pl.ds(i * 16, 16)] = zero16
            return _

        lax.fori_loop(0, RA // 16, fillz, 0)

        pltpu.sync_copy(zbuf, acc.at[pl.ds(s * RA, RA)])

        @pl.when(s == NS - 1)
        def _():
            pltpu.sync_copy(
                zbuf.at[pl.ds(0, TAIL)], acc.at[pl.ds(NS * RA, TAIL)]
            )

        plsc.subcore_barrier()

        def body(i, _):
            b = wid * E_PER_W + i * CHUNK
            pltpu.sync_copy(dst_hbm.at[pl.ds(b, CHUNK)], didx)
            pltpu.sync_copy(ones_v, acc.at[didx], add=True)
            return _

        lax.fori_loop(0, N_CHUNKS, body, 0)
        plsc.subcore_barrier()
        pltpu.sync_copy(acc.at[pl.ds(s * RA, RA)], zbuf)
        pltpu.sync_copy(zbuf, out_hbm.at[pl.ds(c * N_NODES + s * RA, RA)])

        @pl.when(s == NS - 1)
        def _():
            pltpu.sync_copy(acc.at[pl.ds(NS * RA, TAIL)], zbuf.at[pl.ds(0, TAIL)])
            pltpu.sync_copy(
                zbuf.at[pl.ds(0, TAIL)],
                out_hbm.at[pl.ds(c * N_NODES + NS * RA, TAIL)],
            )

    return deg_kernel(dst)


def _edge_scatter_call(hs, src, dst, d):
    @functools.partial(
        pl.kernel,
        out_type=jax.ShapeDtypeStruct((NC * N_NODES, d), jnp.float32),
        mesh=_sc_mesh(),
        scratch_types=[
            pltpu.VMEM((CHUNK,), jnp.int32),
            pltpu.VMEM((CHUNK,), jnp.int32),
            pltpu.VMEM((CHUNK, d), jnp.float32),
            pltpu.VMEM((ZR, d), jnp.float32),
            pltpu.VMEM_SHARED((N_NODES, d), jnp.float32),
            pltpu.SemaphoreType.DMA,
        ],
    )
    def scat_kernel(hs_hbm, src_hbm, dst_hbm, out_hbm, sidx, didx, rows, zbuf, acc, sem):
        c = lax.axis_index("c")
        s = lax.axis_index("s")
        wid = s * NC + c
        zero16 = jnp.zeros((16,), jnp.float32)

        def fillz(i, _):
            for j in range(d // 16):
                zbuf[i, pl.ds(j * 16, 16)] = zero16
            return _

        lax.fori_loop(0, ZR, fillz, 0)

        def zacc(r, _):
            pltpu.sync_copy(zbuf, acc.at[pl.ds(s * RA + r * ZR, ZR)])
            return _

        lax.fori_loop(0, RA // ZR, zacc, 0)

        @pl.when(s == NS - 1)
        def _():
            pltpu.sync_copy(
                zbuf.at[pl.ds(0, TAIL)], acc.at[pl.ds(NS * RA, TAIL)]
            )

        plsc.subcore_barrier()

        def body(i, _):
            b = wid * E_PER_W + i * CHUNK
            pltpu.sync_copy(src_hbm.at[pl.ds(b, CHUNK)], sidx)
            pltpu.sync_copy(dst_hbm.at[pl.ds(b, CHUNK)], didx)
            pltpu.async_copy(hs_hbm.at[sidx], rows, sem).wait()
            pltpu.sync_copy(rows, acc.at[didx], add=True)
            return _

        lax.fori_loop(0, N_CHUNKS, body, 0)
        plsc.subcore_barrier()
        pltpu.sync_copy(
            acc.at[pl.ds(s * RA, RA)],
            out_hbm.at[pl.ds(c * N_NODES + s * RA, RA)],
        )

        @pl.when(s == NS - 1)
        def _():
            pltpu.sync_copy(
                acc.at[pl.ds(NS * RA, TAIL)],
                out_hbm.at[pl.ds(c * N_NODES + NS * RA, TAIL)],
            )

    return scat_kernel(hs, src, dst)


def _k1_call(z, W1, d0, d1):
    def body(z_ref, w_ref, d0_ref, d1_ref, hs_ref, dinv_ref):
        deg = 1.0 + d0_ref[:, :] + d1_ref[:, :]
        dinv = lax.rsqrt(deg)
        h = jnp.dot(z_ref[:, :], w_ref[:, :], preferred_element_type=jnp.float32)
        hs_ref[:, :] = h * dinv
        dinv_ref[:, :] = dinv

    return pl.pallas_call(
        body,
        grid=(NB,),
        in_specs=[
            pl.BlockSpec((BR, IN_CH), lambda i: (i, 0)),
            pl.BlockSpec((IN_CH, HID), lambda i: (0, 0)),
            pl.BlockSpec((BR, 1), lambda i: (i, 0)),
            pl.BlockSpec((BR, 1), lambda i: (i, 0)),
        ],
        out_specs=[
            pl.BlockSpec((BR, HID), lambda i: (i, 0)),
            pl.BlockSpec((BR, 1), lambda i: (i, 0)),
        ],
        out_shape=[
            jax.ShapeDtypeStruct((N_NODES, HID), jnp.float32),
            jax.ShapeDtypeStruct((N_NODES, 1), jnp.float32),
        ],
    )(z, W1, d0, d1)


def _k2_call(accp1, hs1, dinv, W2, b1):
    def body(p0_ref, p1_ref, hs_ref, dinv_ref, w_ref, b_ref, o_ref):
        dinv = dinv_ref[:, :]
        t = (p0_ref[:, :] + p1_ref[:, :] + hs_ref[:, :]) * dinv + b_ref[:, :]
        r = jnp.maximum(t, 0.0)
        h2 = jnp.dot(r, w_ref[:, :], preferred_element_type=jnp.float32) * dinv
        o_ref[:, :] = jnp.concatenate(
            [h2, jnp.zeros((BR, HID - OUT), jnp.float32)], axis=1
        )

    return pl.pallas_call(
        body,
        grid=(NB,),
        in_specs=[
            pl.BlockSpec((BR, HID), lambda i: (i, 0)),
            pl.BlockSpec((BR, HID), lambda i: (i + NB, 0)),
            pl.BlockSpec((BR, HID), lambda i: (i, 0)),
            pl.BlockSpec((BR, 1), lambda i: (i, 0)),
            pl.BlockSpec((HID, OUT), lambda i: (0, 0)),
            pl.BlockSpec((1, HID), lambda i: (0, 0)),
        ],
        out_specs=pl.BlockSpec((BR, HID), lambda i: (i, 0)),
        out_shape=jax.ShapeDtypeStruct((N_NODES, HID), jnp.float32),
    )(accp1, accp1, hs1, dinv, W2, b1)


def _k3_call(accp2, hs2, dinv, Wfc, b2, bfc):
    def body(q0_ref, q1_ref, hs_ref, dinv_ref, w_ref, b2_ref, bfc_ref, o_ref):
        acc = q0_ref[:, 0:OUT] + q1_ref[:, 0:OUT] + hs_ref[:, 0:OUT]
        t = acc * dinv_ref[:, :] + b2_ref[:, :]
        o_ref[:, :] = (
            jnp.dot(t, w_ref[:, :], preferred_element_type=jnp.float32) + bfc_ref[:, :]
        )

    return pl.pallas_call(
        body,
        grid=(NB,),
        in_specs=[
            pl.BlockSpec((BR, HID), lambda i: (i, 0)),
            pl.BlockSpec((BR, HID), lambda i: (i + NB, 0)),
            pl.BlockSpec((BR, HID), lambda i: (i, 0)),
            pl.BlockSpec((BR, 1), lambda i: (i, 0)),
            pl.BlockSpec((OUT, FC), lambda i: (0, 0)),
            pl.BlockSpec((1, OUT), lambda i: (0, 0)),
            pl.BlockSpec((1, FC), lambda i: (0, 0)),
        ],
        out_specs=pl.BlockSpec((BR, FC), lambda i: (i, 0)),
        out_shape=jax.ShapeDtypeStruct((N_NODES, FC), jnp.float32),
    )(accp2, accp2, hs2, dinv, Wfc, b2, bfc)


def kernel(z, edge_index, W1, b1, W2, b2, Wfc, bfc):
    ei = edge_index.astype(jnp.int32)
    src = ei[0]
    dst = ei[1]

    degp = _deg_call(dst)
    d0 = degp[:N_NODES].reshape(N_NODES, 1)
    d1 = degp[N_NODES:].reshape(N_NODES, 1)
    hs1, dinv = _k1_call(z, W1, d0, d1)
    accp1 = _edge_scatter_call(hs1, src, dst, HID)
    hs2 = _k2_call(accp1, hs1, dinv, W2, b1.reshape(1, HID))
    accp2 = _edge_scatter_call(hs2, src, dst, HID)
    out = _k3_call(accp2, hs2, dinv, Wfc, b2.reshape(1, OUT), bfc.reshape(1, FC))
    return out


# speedup vs baseline: 25.2691x; 1.9692x over previous
import functools

import jax
import jax.numpy as jnp
from jax import lax
from jax.experimental import pallas as pl
from jax.experimental.pallas import tpu as pltpu
from jax.experimental.pallas import tpu_sc as plsc

N_NODES = 10000
N_EDGES = 320000
IN_CH = 128
HID = 128
OUT = 64
FC = 1024

NC = 2
NS = 16
NW = NC * NS
E_PER_W = N_EDGES // NW
CHUNK = 80
N_CHUNKS = E_PER_W // CHUNK
RA = 624
TAIL = N_NODES - (NS - 1) * RA - RA
ZR = 52
DEGW = 16

BR = 1000
NB = N_NODES // BR


def _sc_mesh():
    return plsc.VectorSubcoreMesh(core_axis_name="c", subcore_axis_name="s")


def _deg_call(dst):
    @functools.partial(
        pl.kernel,
        out_type=jax.ShapeDtypeStruct((NC * N_NODES,), jnp.float32),
        mesh=_sc_mesh(),
        scratch_types=[
            pltpu.VMEM((CHUNK,), jnp.int32),
            pltpu.VMEM((CHUNK,), jnp.float32),
            pltpu.VMEM((RA,), jnp.float32),
            pltpu.VMEM_SHARED((N_NODES,), jnp.float32),
        ],
    )
    def deg_kernel(dst_hbm, out_hbm, didx, ones_v, zbuf, acc):
        c = lax.axis_index("c")
        s = lax.axis_index("s")
        wid = s * NC + c
        one16 = jnp.ones((16,), jnp.float32)
        zero16 = jnp.zeros((16,), jnp.float32)

        def fill(i, _):
            ones_v[pl.ds(i * 16, 16)] = one16
            return _

        lax.fori_loop(0, CHUNK // 16, fill, 0)

        def fillz(i, _):
            zbuf[pl.ds(i * 16, 16)] = zero16
            return _

        lax.fori_loop(0, RA // 16, fillz, 0)

        pltpu.sync_copy(zbuf, acc.at[pl.ds(s * RA, RA)])

        @pl.when(s == NS - 1)
        def _():
            pltpu.sync_copy(
                zbuf.at[pl.ds(0, TAIL)], acc.at[pl.ds(NS * RA, TAIL)]
            )

        plsc.subcore_barrier()

        def body(i, _):
            b = wid * E_PER_W + i * CHUNK
            pltpu.sync_copy(dst_hbm.at[pl.ds(b, CHUNK)], didx)
            pltpu.sync_copy(ones_v, acc.at[didx], add=True)
            return _

        lax.fori_loop(0, N_CHUNKS, body, 0)
        plsc.subcore_barrier()
        pltpu.sync_copy(acc.at[pl.ds(s * RA, RA)], zbuf)
        pltpu.sync_copy(zbuf, out_hbm.at[pl.ds(c * N_NODES + s * RA, RA)])

        @pl.when(s == NS - 1)
        def _():
            pltpu.sync_copy(acc.at[pl.ds(NS * RA, TAIL)], zbuf.at[pl.ds(0, TAIL)])
            pltpu.sync_copy(
                zbuf.at[pl.ds(0, TAIL)],
                out_hbm.at[pl.ds(c * N_NODES + NS * RA, TAIL)],
            )

    return deg_kernel(dst)


def _edge_scatter_call(hs, src, dst, d):
    @functools.partial(
        pl.kernel,
        out_type=jax.ShapeDtypeStruct((NC * N_NODES, d), jnp.float32),
        mesh=_sc_mesh(),
        scratch_types=[
            pltpu.VMEM((E_PER_W,), jnp.int32),
            pltpu.VMEM((E_PER_W,), jnp.int32),
            pltpu.VMEM((CHUNK, d), jnp.float32),
            pltpu.VMEM((CHUNK, d), jnp.float32),
            pltpu.VMEM((ZR, d), jnp.float32),
            pltpu.VMEM_SHARED((N_NODES, d), jnp.float32),
            pltpu.SemaphoreType.DMA,
            pltpu.SemaphoreType.DMA,
        ],
    )
    def scat_kernel(
        hs_hbm, src_hbm, dst_hbm, out_hbm,
        sall, dall, rows0, rows1, zbuf, acc, sem0, sem1,
    ):
        c = lax.axis_index("c")
        s = lax.axis_index("s")
        wid = s * NC + c
        zero16 = jnp.zeros((16,), jnp.float32)
        rows = (rows0, rows1)
        sems = (sem0, sem1)

        def fillz(i, _):
            for j in range(d // 16):
                zbuf[i, pl.ds(j * 16, 16)] = zero16
            return _

        lax.fori_loop(0, ZR, fillz, 0)

        def zacc(r, _):
            pltpu.sync_copy(zbuf, acc.at[pl.ds(s * RA + r * ZR, ZR)])
            return _

        lax.fori_loop(0, RA // ZR, zacc, 0)

        @pl.when(s == NS - 1)
        def _():
            pltpu.sync_copy(
                zbuf.at[pl.ds(0, TAIL)], acc.at[pl.ds(NS * RA, TAIL)]
            )

        eb = wid * E_PER_W
        pltpu.sync_copy(src_hbm.at[pl.ds(eb, E_PER_W)], sall)
        pltpu.sync_copy(dst_hbm.at[pl.ds(eb, E_PER_W)], dall)
        plsc.subcore_barrier()

        def gather_start(j, b):
            pltpu.async_copy(
                hs_hbm.at[sall.at[pl.ds(j * CHUNK, CHUNK)]], rows[b], sems[b]
            )

        def gather_wait(b):
            pltpu.make_async_copy(
                hs_hbm.at[pl.ds(0, CHUNK)], rows[b], sems[b]
            ).wait()

        def scatter(j, b):
            pltpu.sync_copy(
                rows[b], acc.at[dall.at[pl.ds(j * CHUNK, CHUNK)]], add=True
            )

        gather_start(0, 0)

        def pair(r, _):
            for b in (0, 1):
                j = 2 * r + b
                gather_start(j + 1, 1 - b)
                gather_wait(b)
                scatter(j, b)
            return _

        lax.fori_loop(0, (N_CHUNKS - 1) // 2, pair, 0)
        gather_wait(0)
        scatter(N_CHUNKS - 1, 0)
        plsc.subcore_barrier()
        pltpu.sync_copy(
            acc.at[pl.ds(s * RA, RA)],
            out_hbm.at[pl.ds(c * N_NODES + s * RA, RA)],
        )

        @pl.when(s == NS - 1)
        def _():
            pltpu.sync_copy(
                acc.at[pl.ds(NS * RA, TAIL)],
                out_hbm.at[pl.ds(c * N_NODES + NS * RA, TAIL)],
            )

    return scat_kernel(hs, src, dst)


def _k1_call(z, W1, d0, d1):
    def body(z_ref, w_ref, d0_ref, d1_ref, hs_ref, dinv_ref):
        deg = 1.0 + d0_ref[:, :] + d1_ref[:, :]
        dinv = lax.rsqrt(deg)
        h = jnp.dot(z_ref[:, :], w_ref[:, :], preferred_element_type=jnp.float32)
        hs_ref[:, :] = h * dinv
        dinv_ref[:, :] = dinv

    return pl.pallas_call(
        body,
        grid=(NB,),
        in_specs=[
            pl.BlockSpec((BR, IN_CH), lambda i: (i, 0)),
            pl.BlockSpec((IN_CH, HID), lambda i: (0, 0)),
            pl.BlockSpec((BR, 1), lambda i: (i, 0)),
            pl.BlockSpec((BR, 1), lambda i: (i, 0)),
        ],
        out_specs=[
            pl.BlockSpec((BR, HID), lambda i: (i, 0)),
            pl.BlockSpec((BR, 1), lambda i: (i, 0)),
        ],
        out_shape=[
            jax.ShapeDtypeStruct((N_NODES, HID), jnp.float32),
            jax.ShapeDtypeStruct((N_NODES, 1), jnp.float32),
        ],
    )(z, W1, d0, d1)


def _k2_call(accp1, hs1, dinv, W2, b1):
    def body(p0_ref, p1_ref, hs_ref, dinv_ref, w_ref, b_ref, o_ref):
        dinv = dinv_ref[:, :]
        t = (p0_ref[:, :] + p1_ref[:, :] + hs_ref[:, :]) * dinv + b_ref[:, :]
        r = jnp.maximum(t, 0.0)
        h2 = jnp.dot(r, w_ref[:, :], preferred_element_type=jnp.float32) * dinv
        o_ref[:, :] = jnp.concatenate(
            [h2, jnp.zeros((BR, HID - OUT), jnp.float32)], axis=1
        )

    return pl.pallas_call(
        body,
        grid=(NB,),
        in_specs=[
            pl.BlockSpec((BR, HID), lambda i: (i, 0)),
            pl.BlockSpec((BR, HID), lambda i: (i + NB, 0)),
            pl.BlockSpec((BR, HID), lambda i: (i, 0)),
            pl.BlockSpec((BR, 1), lambda i: (i, 0)),
            pl.BlockSpec((HID, OUT), lambda i: (0, 0)),
            pl.BlockSpec((1, HID), lambda i: (0, 0)),
        ],
        out_specs=pl.BlockSpec((BR, HID), lambda i: (i, 0)),
        out_shape=jax.ShapeDtypeStruct((N_NODES, HID), jnp.float32),
    )(accp1, accp1, hs1, dinv, W2, b1)


def _k3_call(accp2, hs2, dinv, Wfc, b2, bfc):
    def body(q0_ref, q1_ref, hs_ref, dinv_ref, w_ref, b2_ref, bfc_ref, o_ref):
        acc = q0_ref[:, 0:OUT] + q1_ref[:, 0:OUT] + hs_ref[:, 0:OUT]
        t = acc * dinv_ref[:, :] + b2_ref[:, :]
        o_ref[:, :] = (
            jnp.dot(t, w_ref[:, :], preferred_element_type=jnp.float32) + bfc_ref[:, :]
        )

    return pl.pallas_call(
        body,
        grid=(NB,),
        in_specs=[
            pl.BlockSpec((BR, HID), lambda i: (i, 0)),
            pl.BlockSpec((BR, HID), lambda i: (i + NB, 0)),
            pl.BlockSpec((BR, HID), lambda i: (i, 0)),
            pl.BlockSpec((BR, 1), lambda i: (i, 0)),
            pl.BlockSpec((OUT, FC), lambda i: (0, 0)),
            pl.BlockSpec((1, OUT), lambda i: (0, 0)),
            pl.BlockSpec((1, FC), lambda i: (0, 0)),
        ],
        out_specs=pl.BlockSpec((BR, FC), lambda i: (i, 0)),
        out_shape=jax.ShapeDtypeStruct((N_NODES, FC), jnp.float32),
    )(accp2, accp2, hs2, dinv, Wfc, b2, bfc)


def kernel(z, edge_index, W1, b1, W2, b2, Wfc, bfc):
    ei = edge_index.astype(jnp.int32)
    src = ei[0]
    dst = ei[1]

    degp = _deg_call(dst)
    d0 = degp[:N_NODES].reshape(N_NODES, 1)
    d1 = degp[N_NODES:].reshape(N_NODES, 1)
    hs1, dinv = _k1_call(z, W1, d0, d1)
    accp1 = _edge_scatter_call(hs1, src, dst, HID)
    hs2 = _k2_call(accp1, hs1, dinv, W2, b1.reshape(1, HID))
    accp2 = _edge_scatter_call(hs2, src, dst, HID)
    out = _k3_call(accp2, hs2, dinv, Wfc, b2.reshape(1, OUT), bfc.reshape(1, FC))
    return out


# speedup vs baseline: 29.5360x; 1.1689x over previous
import functools

import jax
import jax.numpy as jnp
from jax import lax
from jax.experimental import pallas as pl
from jax.experimental.pallas import tpu as pltpu
from jax.experimental.pallas import tpu_sc as plsc

N_NODES = 10000
N_EDGES = 320000
IN_CH = 128
HID = 128
OUT = 64
FC = 1024

NC = 2
NS = 16
NW = NC * NS
E_PER_W = N_EDGES // NW
CHUNK = 80
N_CHUNKS = E_PER_W // CHUNK
RA = 624
TAIL = N_NODES - (NS - 1) * RA - RA
ZR = 52
DEGW = 16

BR = 1000
NB = N_NODES // BR


def _sc_mesh():
    return plsc.VectorSubcoreMesh(core_axis_name="c", subcore_axis_name="s")


def _deg_call(dst):
    @functools.partial(
        pl.kernel,
        out_type=jax.ShapeDtypeStruct((NC * N_NODES,), jnp.float32),
        mesh=_sc_mesh(),
        scratch_types=[
            pltpu.VMEM((E_PER_W,), jnp.int32),
            pltpu.VMEM((CHUNK,), jnp.float32),
            pltpu.VMEM((RA,), jnp.float32),
            pltpu.VMEM_SHARED((N_NODES,), jnp.float32),
            pltpu.SemaphoreType.DMA,
            pltpu.SemaphoreType.DMA,
        ],
    )
    def deg_kernel(dst_hbm, out_hbm, dall, ones_v, zbuf, acc, sem0, sem1):
        c = lax.axis_index("c")
        s = lax.axis_index("s")
        wid = s * NC + c
        one16 = jnp.ones((16,), jnp.float32)
        zero16 = jnp.zeros((16,), jnp.float32)

        def fill(i, _):
            ones_v[pl.ds(i * 16, 16)] = one16
            return _

        lax.fori_loop(0, CHUNK // 16, fill, 0)

        def fillz(i, _):
            zbuf[pl.ds(i * 16, 16)] = zero16
            return _

        lax.fori_loop(0, RA // 16, fillz, 0)

        pltpu.sync_copy(zbuf, acc.at[pl.ds(s * RA, RA)])

        @pl.when(s == NS - 1)
        def _():
            pltpu.sync_copy(
                zbuf.at[pl.ds(0, TAIL)], acc.at[pl.ds(NS * RA, TAIL)]
            )

        pltpu.sync_copy(dst_hbm.at[pl.ds(wid * E_PER_W, E_PER_W)], dall)
        plsc.subcore_barrier()

        sems = (sem0, sem1)

        def scat_start(j, b):
            pltpu.async_copy(
                ones_v, acc.at[dall.at[pl.ds(j * CHUNK, CHUNK)]], sems[b],
                add=True,
            )

        def scat_wait(b):
            pltpu.make_async_copy(ones_v, acc.at[pl.ds(0, CHUNK)], sems[b]).wait()

        scat_start(0, 0)

        def pair(r, _):
            for b in (0, 1):
                j = 2 * r + b
                scat_start(j + 1, 1 - b)
                scat_wait(b)
            return _

        lax.fori_loop(0, (N_CHUNKS - 1) // 2, pair, 0)
        scat_wait(0)
        plsc.subcore_barrier()
        pltpu.sync_copy(acc.at[pl.ds(s * RA, RA)], zbuf)
        pltpu.sync_copy(zbuf, out_hbm.at[pl.ds(c * N_NODES + s * RA, RA)])

        @pl.when(s == NS - 1)
        def _():
            pltpu.sync_copy(acc.at[pl.ds(NS * RA, TAIL)], zbuf.at[pl.ds(0, TAIL)])
            pltpu.sync_copy(
                zbuf.at[pl.ds(0, TAIL)],
                out_hbm.at[pl.ds(c * N_NODES + NS * RA, TAIL)],
            )

    return deg_kernel(dst)


def _edge_scatter_call(hs, src, dst, d):
    @functools.partial(
        pl.kernel,
        out_type=jax.ShapeDtypeStruct((NC * N_NODES, d), jnp.float32),
        mesh=_sc_mesh(),
        scratch_types=[
            pltpu.VMEM((E_PER_W,), jnp.int32),
            pltpu.VMEM((E_PER_W,), jnp.int32),
            pltpu.VMEM((CHUNK, d), jnp.float32),
            pltpu.VMEM((CHUNK, d), jnp.float32),
            pltpu.VMEM((ZR, d), jnp.float32),
            pltpu.VMEM_SHARED((N_NODES, d), jnp.float32),
            pltpu.SemaphoreType.DMA,
            pltpu.SemaphoreType.DMA,
        ],
    )
    def scat_kernel(
        hs_hbm, src_hbm, dst_hbm, out_hbm,
        sall, dall, rows0, rows1, zbuf, acc, sem0, sem1,
    ):
        c = lax.axis_index("c")
        s = lax.axis_index("s")
        wid = s * NC + c
        zero16 = jnp.zeros((16,), jnp.float32)
        rows = (rows0, rows1)
        sems = (sem0, sem1)

        def fillz(i, _):
            for j in range(d // 16):
                zbuf[i, pl.ds(j * 16, 16)] = zero16
            return _

        lax.fori_loop(0, ZR, fillz, 0)

        def zacc(r, _):
            pltpu.sync_copy(zbuf, acc.at[pl.ds(s * RA + r * ZR, ZR)])
            return _

        lax.fori_loop(0, RA // ZR, zacc, 0)

        @pl.when(s == NS - 1)
        def _():
            pltpu.sync_copy(
                zbuf.at[pl.ds(0, TAIL)], acc.at[pl.ds(NS * RA, TAIL)]
            )

        eb = wid * E_PER_W
        pltpu.sync_copy(src_hbm.at[pl.ds(eb, E_PER_W)], sall)
        pltpu.sync_copy(dst_hbm.at[pl.ds(eb, E_PER_W)], dall)
        plsc.subcore_barrier()

        def gather_start(j, b):
            pltpu.async_copy(
                hs_hbm.at[sall.at[pl.ds(j * CHUNK, CHUNK)]], rows[b], sems[b]
            )

        def gather_wait(b):
            pltpu.make_async_copy(
                hs_hbm.at[pl.ds(0, CHUNK)], rows[b], sems[b]
            ).wait()

        def scatter(j, b):
            pltpu.sync_copy(
                rows[b], acc.at[dall.at[pl.ds(j * CHUNK, CHUNK)]], add=True
            )

        gather_start(0, 0)

        def pair(r, _):
            for b in (0, 1):
                j = 2 * r + b
                gather_start(j + 1, 1 - b)
                gather_wait(b)
                scatter(j, b)
            return _

        lax.fori_loop(0, (N_CHUNKS - 1) // 2, pair, 0)
        gather_wait(0)
        scatter(N_CHUNKS - 1, 0)
        plsc.subcore_barrier()
        pltpu.sync_copy(
            acc.at[pl.ds(s * RA, RA)],
            out_hbm.at[pl.ds(c * N_NODES + s * RA, RA)],
        )

        @pl.when(s == NS - 1)
        def _():
            pltpu.sync_copy(
                acc.at[pl.ds(NS * RA, TAIL)],
                out_hbm.at[pl.ds(c * N_NODES + NS * RA, TAIL)],
            )

    return scat_kernel(hs, src, dst)


def _k1a_call(z, W1):
    def body(z_ref, w_ref, h_ref):
        h_ref[:, :] = jnp.dot(
            z_ref[:, :], w_ref[:, :], preferred_element_type=jnp.float32
        )

    return pl.pallas_call(
        body,
        grid=(NB,),
        in_specs=[
            pl.BlockSpec((BR, IN_CH), lambda i: (i, 0)),
            pl.BlockSpec((IN_CH, HID), lambda i: (0, 0)),
        ],
        out_specs=pl.BlockSpec((BR, HID), lambda i: (i, 0)),
        out_shape=jax.ShapeDtypeStruct((N_NODES, HID), jnp.float32),
    )(z, W1)


def _k1b_call(h1, d0, d1):
    def body(h_ref, d0_ref, d1_ref, hs_ref, dinv_ref):
        deg = 1.0 + d0_ref[:, :] + d1_ref[:, :]
        dinv = lax.rsqrt(deg)
        hs_ref[:, :] = h_ref[:, :] * dinv
        dinv_ref[:, :] = dinv

    return pl.pallas_call(
        body,
        grid=(NB,),
        in_specs=[
            pl.BlockSpec((BR, HID), lambda i: (i, 0)),
            pl.BlockSpec((BR, 1), lambda i: (i, 0)),
            pl.BlockSpec((BR, 1), lambda i: (i, 0)),
        ],
        out_specs=[
            pl.BlockSpec((BR, HID), lambda i: (i, 0)),
            pl.BlockSpec((BR, 1), lambda i: (i, 0)),
        ],
        out_shape=[
            jax.ShapeDtypeStruct((N_NODES, HID), jnp.float32),
            jax.ShapeDtypeStruct((N_NODES, 1), jnp.float32),
        ],
    )(h1, d0, d1)


def _k2_call(accp1, hs1, dinv, W2, b1):
    def body(p0_ref, p1_ref, hs_ref, dinv_ref, w_ref, b_ref, o_ref):
        dinv = dinv_ref[:, :]
        t = (p0_ref[:, :] + p1_ref[:, :] + hs_ref[:, :]) * dinv + b_ref[:, :]
        r = jnp.maximum(t, 0.0)
        h2 = jnp.dot(r, w_ref[:, :], preferred_element_type=jnp.float32) * dinv
        o_ref[:, :] = jnp.concatenate(
            [h2, jnp.zeros((BR, HID - OUT), jnp.float32)], axis=1
        )

    return pl.pallas_call(
        body,
        grid=(NB,),
        in_specs=[
            pl.BlockSpec((BR, HID), lambda i: (i, 0)),
            pl.BlockSpec((BR, HID), lambda i: (i + NB, 0)),
            pl.BlockSpec((BR, HID), lambda i: (i, 0)),
            pl.BlockSpec((BR, 1), lambda i: (i, 0)),
            pl.BlockSpec((HID, OUT), lambda i: (0, 0)),
            pl.BlockSpec((1, HID), lambda i: (0, 0)),
        ],
        out_specs=pl.BlockSpec((BR, HID), lambda i: (i, 0)),
        out_shape=jax.ShapeDtypeStruct((N_NODES, HID), jnp.float32),
    )(accp1, accp1, hs1, dinv, W2, b1)


def _k3_call(accp2, hs2, dinv, Wfc, b2, bfc):
    def body(q0_ref, q1_ref, hs_ref, dinv_ref, w_ref, b2_ref, bfc_ref, o_ref):
        acc = q0_ref[:, 0:OUT] + q1_ref[:, 0:OUT] + hs_ref[:, 0:OUT]
        t = acc * dinv_ref[:, :] + b2_ref[:, :]
        o_ref[:, :] = (
            jnp.dot(t, w_ref[:, :], preferred_element_type=jnp.float32) + bfc_ref[:, :]
        )

    return pl.pallas_call(
        body,
        grid=(NB,),
        in_specs=[
            pl.BlockSpec((BR, HID), lambda i: (i, 0)),
            pl.BlockSpec((BR, HID), lambda i: (i + NB, 0)),
            pl.BlockSpec((BR, HID), lambda i: (i, 0)),
            pl.BlockSpec((BR, 1), lambda i: (i, 0)),
            pl.BlockSpec((OUT, FC), lambda i: (0, 0)),
            pl.BlockSpec((1, OUT), lambda i: (0, 0)),
            pl.BlockSpec((1, FC), lambda i: (0, 0)),
        ],
        out_specs=pl.BlockSpec((BR, FC), lambda i: (i, 0)),
        out_shape=jax.ShapeDtypeStruct((N_NODES, FC), jnp.float32),
    )(accp2, accp2, hs2, dinv, Wfc, b2, bfc)


def kernel(z, edge_index, W1, b1, W2, b2, Wfc, bfc):
    ei = edge_index.astype(jnp.int32)
    src = ei[0]
    dst = ei[1]

    degp = _deg_call(dst)
    h1 = _k1a_call(z, W1)
    d0 = degp[:N_NODES].reshape(N_NODES, 1)
    d1 = degp[N_NODES:].reshape(N_NODES, 1)
    hs1, dinv = _k1b_call(h1, d0, d1)
    accp1 = _edge_scatter_call(hs1, src, dst, HID)
    hs2 = _k2_call(accp1, hs1, dinv, W2, b1.reshape(1, HID))
    accp2 = _edge_scatter_call(hs2, src, dst, HID)
    out = _k3_call(accp2, hs2, dinv, Wfc, b2.reshape(1, OUT), bfc.reshape(1, FC))
    return out


# speedup vs baseline: 31.6706x; 1.0723x over previous
import functools

import jax
import jax.numpy as jnp
from jax import lax
from jax.experimental import pallas as pl
from jax.experimental.pallas import tpu as pltpu
from jax.experimental.pallas import tpu_sc as plsc

N_NODES = 10000
N_EDGES = 320000
IN_CH = 128
HID = 128
OUT = 64
FC = 1024

NC = 2
NS = 16
NW = NC * NS
E_PER_W = N_EDGES // NW
CHUNK = 80
N_CHUNKS = E_PER_W // CHUNK
RA = 624
TAIL = N_NODES - (NS - 1) * RA - RA
ZR = 52
DEGW = 16

BR = 1000
NB = N_NODES // BR


def _sc_mesh():
    return plsc.VectorSubcoreMesh(core_axis_name="c", subcore_axis_name="s")


def _deg_call(dst):
    @functools.partial(
        pl.kernel,
        out_type=jax.ShapeDtypeStruct((NC * N_NODES,), jnp.float32),
        mesh=_sc_mesh(),
        scratch_types=[
            pltpu.VMEM((E_PER_W,), jnp.int32),
            pltpu.VMEM((CHUNK,), jnp.float32),
            pltpu.VMEM((RA,), jnp.float32),
            pltpu.VMEM_SHARED((N_NODES,), jnp.float32),
            pltpu.SemaphoreType.DMA,
            pltpu.SemaphoreType.DMA,
        ],
    )
    def deg_kernel(dst_hbm, out_hbm, dall, ones_v, zbuf, acc, sem0, sem1):
        c = lax.axis_index("c")
        s = lax.axis_index("s")
        wid = s * NC + c
        one16 = jnp.ones((16,), jnp.float32)
        zero16 = jnp.zeros((16,), jnp.float32)

        def fill(i, _):
            ones_v[pl.ds(i * 16, 16)] = one16
            return _

        lax.fori_loop(0, CHUNK // 16, fill, 0)

        def fillz(i, _):
            zbuf[pl.ds(i * 16, 16)] = zero16
            return _

        lax.fori_loop(0, RA // 16, fillz, 0)

        pltpu.sync_copy(zbuf, acc.at[pl.ds(s * RA, RA)])

        @pl.when(s == NS - 1)
        def _():
            pltpu.sync_copy(
                zbuf.at[pl.ds(0, TAIL)], acc.at[pl.ds(NS * RA, TAIL)]
            )

        pltpu.sync_copy(dst_hbm.at[pl.ds(wid * E_PER_W, E_PER_W)], dall)
        plsc.subcore_barrier()

        sems = (sem0, sem1)

        def scat_start(j, b):
            pltpu.async_copy(
                ones_v, acc.at[dall.at[pl.ds(j * CHUNK, CHUNK)]], sems[b],
                add=True,
            )

        def scat_wait(b):
            pltpu.make_async_copy(ones_v, acc.at[pl.ds(0, CHUNK)], sems[b]).wait()

        scat_start(0, 0)

        def pair(r, _):
            for b in (0, 1):
                j = 2 * r + b
                scat_start(j + 1, 1 - b)
                scat_wait(b)
            return _

        lax.fori_loop(0, (N_CHUNKS - 1) // 2, pair, 0)
        scat_wait(0)
        plsc.subcore_barrier()
        pltpu.sync_copy(acc.at[pl.ds(s * RA, RA)], zbuf)
        pltpu.sync_copy(zbuf, out_hbm.at[pl.ds(c * N_NODES + s * RA, RA)])

        @pl.when(s == NS - 1)
        def _():
            pltpu.sync_copy(acc.at[pl.ds(NS * RA, TAIL)], zbuf.at[pl.ds(0, TAIL)])
            pltpu.sync_copy(
                zbuf.at[pl.ds(0, TAIL)],
                out_hbm.at[pl.ds(c * N_NODES + NS * RA, TAIL)],
            )

    return deg_kernel(dst)


def _edge_scatter_call(hs, src, dst, d, tiled=True):
    @functools.partial(
        pl.kernel,
        out_type=jax.ShapeDtypeStruct((NC * N_NODES, d), jnp.float32),
        mesh=_sc_mesh(),
        compiler_params=pltpu.CompilerParams(use_tc_tiling_on_sc=tiled),
        scratch_types=[
            pltpu.VMEM((E_PER_W,), jnp.int32),
            pltpu.VMEM((E_PER_W,), jnp.int32),
            pltpu.VMEM((CHUNK, d), jnp.float32),
            pltpu.VMEM((CHUNK, d), jnp.float32),
            pltpu.VMEM((ZR, d), jnp.float32),
            pltpu.VMEM_SHARED((N_NODES, d), jnp.float32),
            pltpu.SemaphoreType.DMA,
            pltpu.SemaphoreType.DMA,
        ],
    )
    def scat_kernel(
        hs_hbm, src_hbm, dst_hbm, out_hbm,
        sall, dall, rows0, rows1, zbuf, acc, sem0, sem1,
    ):
        c = lax.axis_index("c")
        s = lax.axis_index("s")
        wid = s * NC + c
        zero16 = jnp.zeros((16,), jnp.float32)
        rows = (rows0, rows1)
        sems = (sem0, sem1)

        def fillz(i, _):
            for j in range(d // 16):
                zbuf[i, pl.ds(j * 16, 16)] = zero16
            return _

        lax.fori_loop(0, ZR, fillz, 0)

        def zacc(r, _):
            pltpu.sync_copy(zbuf, acc.at[pl.ds(s * RA + r * ZR, ZR)])
            return _

        lax.fori_loop(0, RA // ZR, zacc, 0)

        @pl.when(s == NS - 1)
        def _():
            pltpu.sync_copy(
                zbuf.at[pl.ds(0, TAIL)], acc.at[pl.ds(NS * RA, TAIL)]
            )

        eb = wid * E_PER_W
        pltpu.sync_copy(src_hbm.at[pl.ds(eb, E_PER_W)], sall)
        pltpu.sync_copy(dst_hbm.at[pl.ds(eb, E_PER_W)], dall)
        plsc.subcore_barrier()

        def gather_start(j, b):
            pltpu.async_copy(
                hs_hbm.at[sall.at[pl.ds(j * CHUNK, CHUNK)]], rows[b], sems[b]
            )

        def gather_wait(b):
            pltpu.make_async_copy(
                hs_hbm.at[pl.ds(0, CHUNK)], rows[b], sems[b]
            ).wait()

        def scatter(j, b):
            pltpu.sync_copy(
                rows[b], acc.at[dall.at[pl.ds(j * CHUNK, CHUNK)]], add=True
            )

        gather_start(0, 0)

        def pair(r, _):
            for b in (0, 1):
                j = 2 * r + b
                gather_start(j + 1, 1 - b)
                gather_wait(b)
                scatter(j, b)
            return _

        lax.fori_loop(0, (N_CHUNKS - 1) // 2, pair, 0)
        gather_wait(0)
        scatter(N_CHUNKS - 1, 0)
        plsc.subcore_barrier()
        pltpu.sync_copy(
            acc.at[pl.ds(s * RA, RA)],
            out_hbm.at[pl.ds(c * N_NODES + s * RA, RA)],
        )

        @pl.when(s == NS - 1)
        def _():
            pltpu.sync_copy(
                acc.at[pl.ds(NS * RA, TAIL)],
                out_hbm.at[pl.ds(c * N_NODES + NS * RA, TAIL)],
            )

    return scat_kernel(hs, src, dst)


def _k1a_call(z, W1):
    def body(z_ref, w_ref, h_ref):
        h_ref[:, :] = jnp.dot(
            z_ref[:, :], w_ref[:, :], preferred_element_type=jnp.float32
        )

    return pl.pallas_call(
        body,
        grid=(NB,),
        in_specs=[
            pl.BlockSpec((BR, IN_CH), lambda i: (i, 0)),
            pl.BlockSpec((IN_CH, HID), lambda i: (0, 0)),
        ],
        out_specs=pl.BlockSpec((BR, HID), lambda i: (i, 0)),
        out_shape=jax.ShapeDtypeStruct((N_NODES, HID), jnp.float32),
    )(z, W1)


def _k1b_call(h1, d0, d1):
    def body(h_ref, d0_ref, d1_ref, hs_ref, dinv_ref):
        deg = 1.0 + d0_ref[:, :] + d1_ref[:, :]
        dinv = lax.rsqrt(deg)
        hs_ref[:, :] = h_ref[:, :] * dinv
        dinv_ref[:, :] = dinv

    return pl.pallas_call(
        body,
        grid=(NB,),
        in_specs=[
            pl.BlockSpec((BR, HID), lambda i: (i, 0)),
            pl.BlockSpec((BR, 1), lambda i: (i, 0)),
            pl.BlockSpec((BR, 1), lambda i: (i, 0)),
        ],
        out_specs=[
            pl.BlockSpec((BR, HID), lambda i: (i, 0)),
            pl.BlockSpec((BR, 1), lambda i: (i, 0)),
        ],
        out_shape=[
            jax.ShapeDtypeStruct((N_NODES, HID), jnp.float32),
            jax.ShapeDtypeStruct((N_NODES, 1), jnp.float32),
        ],
    )(h1, d0, d1)


def _k2_call(accp1, hs1, dinv, W2, b1):
    def body(p0_ref, p1_ref, hs_ref, dinv_ref, w_ref, b_ref, o_ref):
        dinv = dinv_ref[:, :]
        t = (p0_ref[:, :] + p1_ref[:, :] + hs_ref[:, :]) * dinv + b_ref[:, :]
        r = jnp.maximum(t, 0.0)
        o_ref[:, :] = jnp.dot(r, w_ref[:, :], preferred_element_type=jnp.float32) * dinv

    return pl.pallas_call(
        body,
        grid=(NB,),
        in_specs=[
            pl.BlockSpec((BR, HID), lambda i: (i, 0)),
            pl.BlockSpec((BR, HID), lambda i: (i + NB, 0)),
            pl.BlockSpec((BR, HID), lambda i: (i, 0)),
            pl.BlockSpec((BR, 1), lambda i: (i, 0)),
            pl.BlockSpec((HID, OUT), lambda i: (0, 0)),
            pl.BlockSpec((1, HID), lambda i: (0, 0)),
        ],
        out_specs=pl.BlockSpec((BR, OUT), lambda i: (i, 0)),
        out_shape=jax.ShapeDtypeStruct((N_NODES, OUT), jnp.float32),
    )(accp1, accp1, hs1, dinv, W2, b1)


def _k3_call(accp2, hs2, dinv, Wfc, b2, bfc):
    def body(q0_ref, q1_ref, hs_ref, dinv_ref, w_ref, b2_ref, bfc_ref, o_ref):
        acc = q0_ref[:, :] + q1_ref[:, :] + hs_ref[:, :]
        t = acc * dinv_ref[:, :] + b2_ref[:, :]
        o_ref[:, :] = (
            jnp.dot(t, w_ref[:, :], preferred_element_type=jnp.float32) + bfc_ref[:, :]
        )

    return pl.pallas_call(
        body,
        grid=(NB,),
        in_specs=[
            pl.BlockSpec((BR, OUT), lambda i: (i, 0)),
            pl.BlockSpec((BR, OUT), lambda i: (i + NB, 0)),
            pl.BlockSpec((BR, OUT), lambda i: (i, 0)),
            pl.BlockSpec((BR, 1), lambda i: (i, 0)),
            pl.BlockSpec((OUT, FC), lambda i: (0, 0)),
            pl.BlockSpec((1, OUT), lambda i: (0, 0)),
            pl.BlockSpec((1, FC), lambda i: (0, 0)),
        ],
        out_specs=pl.BlockSpec((BR, FC), lambda i: (i, 0)),
        out_shape=jax.ShapeDtypeStruct((N_NODES, FC), jnp.float32),
    )(accp2, accp2, hs2, dinv, Wfc, b2, bfc)


def kernel(z, edge_index, W1, b1, W2, b2, Wfc, bfc):
    ei = edge_index.astype(jnp.int32)
    src = ei[0]
    dst = ei[1]

    degp = _deg_call(dst)
    h1 = _k1a_call(z, W1)
    d0 = degp[:N_NODES].reshape(N_NODES, 1)
    d1 = degp[N_NODES:].reshape(N_NODES, 1)
    hs1, dinv = _k1b_call(h1, d0, d1)
    accp1 = _edge_scatter_call(hs1, src, dst, HID)
    hs2 = _k2_call(accp1, hs1, dinv, W2, b1.reshape(1, HID))
    accp2 = _edge_scatter_call(hs2, src, dst, OUT, tiled=False)
    out = _k3_call(accp2, hs2, dinv, Wfc, b2.reshape(1, OUT), bfc.reshape(1, FC))
    return out


# speedup vs baseline: 32.0253x; 1.0112x over previous
import functools

import jax
import jax.numpy as jnp
from jax import lax
from jax.experimental import pallas as pl
from jax.experimental.pallas import tpu as pltpu
from jax.experimental.pallas import tpu_sc as plsc

N_NODES = 10000
N_EDGES = 320000
IN_CH = 128
HID = 128
OUT = 64
FC = 1024

NC = 2
NS = 16
NW = NC * NS
E_PER_W = N_EDGES // NW
CHUNK = 80
N_CHUNKS = E_PER_W // CHUNK
RA = 624
TAIL = N_NODES - (NS - 1) * RA - RA
ZR = 52
DEGW = 16

BR = 1000
NB = N_NODES // BR


def _sc_mesh():
    return plsc.VectorSubcoreMesh(core_axis_name="c", subcore_axis_name="s")


def _deg_call(dst):
    @functools.partial(
        pl.kernel,
        out_type=jax.ShapeDtypeStruct((NC * N_NODES,), jnp.float32),
        mesh=_sc_mesh(),
        scratch_types=[
            pltpu.VMEM((E_PER_W,), jnp.int32),
            pltpu.VMEM((CHUNK,), jnp.float32),
            pltpu.VMEM((RA,), jnp.float32),
            pltpu.VMEM_SHARED((N_NODES,), jnp.float32),
            pltpu.SemaphoreType.DMA,
            pltpu.SemaphoreType.DMA,
        ],
    )
    def deg_kernel(dst_hbm, out_hbm, dall, ones_v, zbuf, acc, sem0, sem1):
        c = lax.axis_index("c")
        s = lax.axis_index("s")
        wid = s * NC + c
        one16 = jnp.ones((16,), jnp.float32)
        zero16 = jnp.zeros((16,), jnp.float32)

        def fill(i, _):
            ones_v[pl.ds(i * 16, 16)] = one16
            return _

        lax.fori_loop(0, CHUNK // 16, fill, 0)

        def fillz(i, _):
            zbuf[pl.ds(i * 16, 16)] = zero16
            return _

        lax.fori_loop(0, RA // 16, fillz, 0)

        pltpu.sync_copy(zbuf, acc.at[pl.ds(s * RA, RA)])

        @pl.when(s == NS - 1)
        def _():
            pltpu.sync_copy(
                zbuf.at[pl.ds(0, TAIL)], acc.at[pl.ds(NS * RA, TAIL)]
            )

        pltpu.sync_copy(dst_hbm.at[pl.ds(wid * E_PER_W, E_PER_W)], dall)
        plsc.subcore_barrier()

        sems = (sem0, sem1)

        def scat_start(j, b):
            pltpu.async_copy(
                ones_v, acc.at[dall.at[pl.ds(j * CHUNK, CHUNK)]], sems[b],
                add=True,
            )

        def scat_wait(b):
            pltpu.make_async_copy(ones_v, acc.at[pl.ds(0, CHUNK)], sems[b]).wait()

        scat_start(0, 0)

        def pair(r, _):
            for b in (0, 1):
                j = 2 * r + b
                scat_start(j + 1, 1 - b)
                scat_wait(b)
            return _

        lax.fori_loop(0, (N_CHUNKS - 1) // 2, pair, 0)
        scat_wait(0)
        plsc.subcore_barrier()
        pltpu.sync_copy(acc.at[pl.ds(s * RA, RA)], zbuf)
        pltpu.sync_copy(zbuf, out_hbm.at[pl.ds(c * N_NODES + s * RA, RA)])

        @pl.when(s == NS - 1)
        def _():
            pltpu.sync_copy(acc.at[pl.ds(NS * RA, TAIL)], zbuf.at[pl.ds(0, TAIL)])
            pltpu.sync_copy(
                zbuf.at[pl.ds(0, TAIL)],
                out_hbm.at[pl.ds(c * N_NODES + NS * RA, TAIL)],
            )

    return deg_kernel(dst)


def _edge_scatter_call(hs, src, dst, d, tiled=True, stage=False):
    scratch = [
        pltpu.VMEM((E_PER_W,), jnp.int32),
        pltpu.VMEM((E_PER_W,), jnp.int32),
        pltpu.VMEM((CHUNK, d), jnp.float32),
        pltpu.VMEM((CHUNK, d), jnp.float32),
        pltpu.VMEM((ZR, d), jnp.float32),
        pltpu.VMEM_SHARED((N_NODES, d), jnp.float32),
    ]
    if stage:
        scratch.append(pltpu.VMEM_SHARED((N_NODES, d), jnp.float32))
    scratch += [pltpu.SemaphoreType.DMA, pltpu.SemaphoreType.DMA]

    @functools.partial(
        pl.kernel,
        out_type=jax.ShapeDtypeStruct((NC * N_NODES, d), jnp.float32),
        mesh=_sc_mesh(),
        compiler_params=pltpu.CompilerParams(use_tc_tiling_on_sc=tiled),
        scratch_types=scratch,
    )
    def scat_kernel(hs_hbm, src_hbm, dst_hbm, out_hbm, *scr):
        if stage:
            sall, dall, rows0, rows1, zbuf, acc, hs_spm, sem0, sem1 = scr
            g_src = hs_spm
        else:
            sall, dall, rows0, rows1, zbuf, acc, sem0, sem1 = scr
            g_src = hs_hbm
        c = lax.axis_index("c")
        s = lax.axis_index("s")
        wid = s * NC + c
        zero16 = jnp.zeros((16,), jnp.float32)
        rows = (rows0, rows1)
        sems = (sem0, sem1)

        if stage:
            pltpu.sync_copy(
                hs_hbm.at[pl.ds(s * RA, RA)], hs_spm.at[pl.ds(s * RA, RA)]
            )

            @pl.when(s == NS - 1)
            def _():
                pltpu.sync_copy(
                    hs_hbm.at[pl.ds(NS * RA, TAIL)],
                    hs_spm.at[pl.ds(NS * RA, TAIL)],
                )

        def fillz(i, _):
            for j in range(d // 16):
                zbuf[i, pl.ds(j * 16, 16)] = zero16
            return _

        lax.fori_loop(0, ZR, fillz, 0)

        def zacc(r, _):
            pltpu.sync_copy(zbuf, acc.at[pl.ds(s * RA + r * ZR, ZR)])
            return _

        lax.fori_loop(0, RA // ZR, zacc, 0)

        @pl.when(s == NS - 1)
        def _():
            pltpu.sync_copy(
                zbuf.at[pl.ds(0, TAIL)], acc.at[pl.ds(NS * RA, TAIL)]
            )

        eb = wid * E_PER_W
        pltpu.sync_copy(src_hbm.at[pl.ds(eb, E_PER_W)], sall)
        pltpu.sync_copy(dst_hbm.at[pl.ds(eb, E_PER_W)], dall)
        plsc.subcore_barrier()

        def gather_start(j, b):
            pltpu.async_copy(
                g_src.at[sall.at[pl.ds(j * CHUNK, CHUNK)]], rows[b], sems[b]
            )

        def gather_wait(b):
            pltpu.make_async_copy(
                hs_hbm.at[pl.ds(0, CHUNK)], rows[b], sems[b]
            ).wait()

        def scatter(j, b):
            pltpu.sync_copy(
                rows[b], acc.at[dall.at[pl.ds(j * CHUNK, CHUNK)]], add=True
            )

        gather_start(0, 0)

        def pair(r, _):
            for b in (0, 1):
                j = 2 * r + b
                gather_start(j + 1, 1 - b)
                gather_wait(b)
                scatter(j, b)
            return _

        lax.fori_loop(0, (N_CHUNKS - 1) // 2, pair, 0)
        gather_wait(0)
        scatter(N_CHUNKS - 1, 0)
        plsc.subcore_barrier()
        pltpu.sync_copy(
            acc.at[pl.ds(s * RA, RA)],
            out_hbm.at[pl.ds(c * N_NODES + s * RA, RA)],
        )

        @pl.when(s == NS - 1)
        def _():
            pltpu.sync_copy(
                acc.at[pl.ds(NS * RA, TAIL)],
                out_hbm.at[pl.ds(c * N_NODES + NS * RA, TAIL)],
            )

    return scat_kernel(hs, src, dst)


def _k1a_call(z, W1):
    def body(z_ref, w_ref, h_ref):
        h_ref[:, :] = jnp.dot(
            z_ref[:, :], w_ref[:, :], preferred_element_type=jnp.float32
        )

    return pl.pallas_call(
        body,
        grid=(NB,),
        in_specs=[
            pl.BlockSpec((BR, IN_CH), lambda i: (i, 0)),
            pl.BlockSpec((IN_CH, HID), lambda i: (0, 0)),
        ],
        out_specs=pl.BlockSpec((BR, HID), lambda i: (i, 0)),
        out_shape=jax.ShapeDtypeStruct((N_NODES, HID), jnp.float32),
    )(z, W1)


def _k1b_call(h1, d0, d1):
    def body(h_ref, d0_ref, d1_ref, hs_ref, dinv_ref):
        deg = 1.0 + d0_ref[:, :] + d1_ref[:, :]
        dinv = lax.rsqrt(deg)
        hs_ref[:, :] = h_ref[:, :] * dinv
        dinv_ref[:, :] = dinv

    return pl.pallas_call(
        body,
        grid=(NB,),
        in_specs=[
            pl.BlockSpec((BR, HID), lambda i: (i, 0)),
            pl.BlockSpec((BR, 1), lambda i: (i, 0)),
            pl.BlockSpec((BR, 1), lambda i: (i, 0)),
        ],
        out_specs=[
            pl.BlockSpec((BR, HID), lambda i: (i, 0)),
            pl.BlockSpec((BR, 1), lambda i: (i, 0)),
        ],
        out_shape=[
            jax.ShapeDtypeStruct((N_NODES, HID), jnp.float32),
            jax.ShapeDtypeStruct((N_NODES, 1), jnp.float32),
        ],
    )(h1, d0, d1)


def _k2_call(accp1, hs1, dinv, W2, b1):
    def body(p0_ref, p1_ref, hs_ref, dinv_ref, w_ref, b_ref, o_ref):
        dinv = dinv_ref[:, :]
        t = (p0_ref[:, :] + p1_ref[:, :] + hs_ref[:, :]) * dinv + b_ref[:, :]
        r = jnp.maximum(t, 0.0)
        o_ref[:, :] = jnp.dot(r, w_ref[:, :], preferred_element_type=jnp.float32) * dinv

    return pl.pallas_call(
        body,
        grid=(NB,),
        in_specs=[
            pl.BlockSpec((BR, HID), lambda i: (i, 0)),
            pl.BlockSpec((BR, HID), lambda i: (i + NB, 0)),
            pl.BlockSpec((BR, HID), lambda i: (i, 0)),
            pl.BlockSpec((BR, 1), lambda i: (i, 0)),
            pl.BlockSpec((HID, OUT), lambda i: (0, 0)),
            pl.BlockSpec((1, HID), lambda i: (0, 0)),
        ],
        out_specs=pl.BlockSpec((BR, OUT), lambda i: (i, 0)),
        out_shape=jax.ShapeDtypeStruct((N_NODES, OUT), jnp.float32),
    )(accp1, accp1, hs1, dinv, W2, b1)


def _k3_call(accp2, hs2, dinv, Wfc, b2, bfc):
    def body(q0_ref, q1_ref, hs_ref, dinv_ref, w_ref, b2_ref, bfc_ref, o_ref):
        acc = q0_ref[:, :] + q1_ref[:, :] + hs_ref[:, :]
        t = acc * dinv_ref[:, :] + b2_ref[:, :]
        o_ref[:, :] = (
            jnp.dot(t, w_ref[:, :], preferred_element_type=jnp.float32) + bfc_ref[:, :]
        )

    return pl.pallas_call(
        body,
        grid=(NB,),
        in_specs=[
            pl.BlockSpec((BR, OUT), lambda i: (i, 0)),
            pl.BlockSpec((BR, OUT), lambda i: (i + NB, 0)),
            pl.BlockSpec((BR, OUT), lambda i: (i, 0)),
            pl.BlockSpec((BR, 1), lambda i: (i, 0)),
            pl.BlockSpec((OUT, FC), lambda i: (0, 0)),
            pl.BlockSpec((1, OUT), lambda i: (0, 0)),
            pl.BlockSpec((1, FC), lambda i: (0, 0)),
        ],
        out_specs=pl.BlockSpec((BR, FC), lambda i: (i, 0)),
        out_shape=jax.ShapeDtypeStruct((N_NODES, FC), jnp.float32),
    )(accp2, accp2, hs2, dinv, Wfc, b2, bfc)


def kernel(z, edge_index, W1, b1, W2, b2, Wfc, bfc):
    ei = edge_index.astype(jnp.int32)
    src = ei[0]
    dst = ei[1]

    degp = _deg_call(dst)
    h1 = _k1a_call(z, W1)
    d0 = degp[:N_NODES].reshape(N_NODES, 1)
    d1 = degp[N_NODES:].reshape(N_NODES, 1)
    hs1, dinv = _k1b_call(h1, d0, d1)
    accp1 = _edge_scatter_call(hs1, src, dst, HID)
    hs2 = _k2_call(accp1, hs1, dinv, W2, b1.reshape(1, HID))
    accp2 = _edge_scatter_call(hs2, src, dst, OUT, tiled=False, stage=True)
    out = _k3_call(accp2, hs2, dinv, Wfc, b2.reshape(1, OUT), bfc.reshape(1, FC))
    return out


# speedup vs baseline: 32.0506x; 1.0008x over previous
import functools

import jax
import jax.numpy as jnp
from jax import lax
from jax.experimental import pallas as pl
from jax.experimental.pallas import tpu as pltpu
from jax.experimental.pallas import tpu_sc as plsc

N_NODES = 10000
N_EDGES = 320000
IN_CH = 128
HID = 128
OUT = 64
FC = 1024

NC = 2
NS = 16
NW = NC * NS
E_PER_W = N_EDGES // NW
CHUNK = 80
N_CHUNKS = E_PER_W // CHUNK
RA = 624
TAIL = N_NODES - (NS - 1) * RA - RA
ZR = 52

BR = 1000
NB = N_NODES // BR


def _sc_mesh():
    return plsc.VectorSubcoreMesh(core_axis_name="c", subcore_axis_name="s")


def _deg_call(dst):
    @functools.partial(
        pl.kernel,
        out_type=jax.ShapeDtypeStruct((NC * N_NODES,), jnp.float32),
        mesh=_sc_mesh(),
        scratch_types=[
            pltpu.VMEM((E_PER_W,), jnp.int32),
            pltpu.VMEM((CHUNK,), jnp.float32),
            pltpu.VMEM((RA,), jnp.float32),
            pltpu.VMEM_SHARED((N_NODES,), jnp.float32),
            pltpu.SemaphoreType.DMA,
            pltpu.SemaphoreType.DMA,
        ],
    )
    def deg_kernel(dst_hbm, out_hbm, dall, ones_v, zbuf, acc, sem0, sem1):
        c = lax.axis_index("c")
        s = lax.axis_index("s")
        wid = s * NC + c
        one16 = jnp.ones((16,), jnp.float32)
        zero16 = jnp.zeros((16,), jnp.float32)

        def fill(i, _):
            ones_v[pl.ds(i * 16, 16)] = one16
            return _

        lax.fori_loop(0, CHUNK // 16, fill, 0)

        def fillz(i, _):
            zbuf[pl.ds(i * 16, 16)] = zero16
            return _

        lax.fori_loop(0, RA // 16, fillz, 0)

        pltpu.sync_copy(zbuf, acc.at[pl.ds(s * RA, RA)])

        @pl.when(s == NS - 1)
        def _():
            pltpu.sync_copy(
                zbuf.at[pl.ds(0, TAIL)], acc.at[pl.ds(NS * RA, TAIL)]
            )

        pltpu.sync_copy(dst_hbm.at[pl.ds(wid * E_PER_W, E_PER_W)], dall)
        plsc.subcore_barrier()

        sems = (sem0, sem1)

        def scat_start(j, b):
            pltpu.async_copy(
                ones_v, acc.at[dall.at[pl.ds(j * CHUNK, CHUNK)]], sems[b],
                add=True,
            )

        def scat_wait(b):
            pltpu.make_async_copy(ones_v, acc.at[pl.ds(0, CHUNK)], sems[b]).wait()

        scat_start(0, 0)

        def pair(r, _):
            for b in (0, 1):
                j = 2 * r + b
                scat_start(j + 1, 1 - b)
                scat_wait(b)
            return _

        lax.fori_loop(0, (N_CHUNKS - 1) // 2, pair, 0)
        scat_wait(0)
        plsc.subcore_barrier()
        pltpu.sync_copy(acc.at[pl.ds(s * RA, RA)], zbuf)
        pltpu.sync_copy(zbuf, out_hbm.at[pl.ds(c * N_NODES + s * RA, RA)])

        @pl.when(s == NS - 1)
        def _():
            pltpu.sync_copy(acc.at[pl.ds(NS * RA, TAIL)], zbuf.at[pl.ds(0, TAIL)])
            pltpu.sync_copy(
                zbuf.at[pl.ds(0, TAIL)],
                out_hbm.at[pl.ds(c * N_NODES + NS * RA, TAIL)],
            )

    return deg_kernel(dst)


def _edge_scatter_call(hs, src, dst, d, tiled=True, stage=False):
    scratch = [
        pltpu.VMEM((E_PER_W,), jnp.int32),
        pltpu.VMEM((E_PER_W,), jnp.int32),
        pltpu.VMEM((CHUNK, d), jnp.float32),
        pltpu.VMEM((CHUNK, d), jnp.float32),
        pltpu.VMEM((ZR, d), jnp.float32),
        pltpu.VMEM_SHARED((N_NODES, d), jnp.float32),
    ]
    if stage:
        scratch.append(pltpu.VMEM_SHARED((N_NODES, d), jnp.float32))
    scratch += [pltpu.SemaphoreType.DMA, pltpu.SemaphoreType.DMA]

    @functools.partial(
        pl.kernel,
        out_type=jax.ShapeDtypeStruct((NC * N_NODES, d), jnp.float32),
        mesh=_sc_mesh(),
        compiler_params=pltpu.CompilerParams(use_tc_tiling_on_sc=tiled),
        scratch_types=scratch,
    )
    def scat_kernel(hs_hbm, src_hbm, dst_hbm, out_hbm, *scr):
        if stage:
            sall, dall, rows0, rows1, zbuf, acc, hs_spm, sem0, sem1 = scr
            g_src = hs_spm
        else:
            sall, dall, rows0, rows1, zbuf, acc, sem0, sem1 = scr
            g_src = hs_hbm
        c = lax.axis_index("c")
        s = lax.axis_index("s")
        wid = s * NC + c
        zero16 = jnp.zeros((16,), jnp.float32)
        rows = (rows0, rows1)
        sems = (sem0, sem1)

        if stage:
            pltpu.sync_copy(
                hs_hbm.at[pl.ds(s * RA, RA)], hs_spm.at[pl.ds(s * RA, RA)]
            )

            @pl.when(s == NS - 1)
            def _():
                pltpu.sync_copy(
                    hs_hbm.at[pl.ds(NS * RA, TAIL)],
                    hs_spm.at[pl.ds(NS * RA, TAIL)],
                )

        def fillz(i, _):
            for j in range(d // 16):
                zbuf[i, pl.ds(j * 16, 16)] = zero16
            return _

        lax.fori_loop(0, ZR, fillz, 0)

        def zacc(r, _):
            pltpu.sync_copy(zbuf, acc.at[pl.ds(s * RA + r * ZR, ZR)])
            return _

        lax.fori_loop(0, RA // ZR, zacc, 0)

        @pl.when(s == NS - 1)
        def _():
            pltpu.sync_copy(
                zbuf.at[pl.ds(0, TAIL)], acc.at[pl.ds(NS * RA, TAIL)]
            )

        eb = wid * E_PER_W
        pltpu.sync_copy(src_hbm.at[pl.ds(eb, E_PER_W)], sall)
        pltpu.sync_copy(dst_hbm.at[pl.ds(eb, E_PER_W)], dall)
        plsc.subcore_barrier()

        def gather_start(j, b):
            pltpu.async_copy(
                g_src.at[sall.at[pl.ds(j * CHUNK, CHUNK)]], rows[b], sems[b]
            )

        def gather_wait(b):
            pltpu.make_async_copy(
                hs_hbm.at[pl.ds(0, CHUNK)], rows[b], sems[b]
            ).wait()

        def scatter(j, b):
            pltpu.sync_copy(
                rows[b], acc.at[dall.at[pl.ds(j * CHUNK, CHUNK)]], add=True
            )

        gather_start(0, 0)

        def pair(r, _):
            for b in (0, 1):
                j = 2 * r + b
                gather_start(j + 1, 1 - b)
                gather_wait(b)
                scatter(j, b)
            return _

        lax.fori_loop(0, (N_CHUNKS - 1) // 2, pair, 0)
        gather_wait(0)
        scatter(N_CHUNKS - 1, 0)
        plsc.subcore_barrier()
        pltpu.sync_copy(
            acc.at[pl.ds(s * RA, RA)],
            out_hbm.at[pl.ds(c * N_NODES + s * RA, RA)],
        )

        @pl.when(s == NS - 1)
        def _():
            pltpu.sync_copy(
                acc.at[pl.ds(NS * RA, TAIL)],
                out_hbm.at[pl.ds(c * N_NODES + NS * RA, TAIL)],
            )

    return scat_kernel(hs, src, dst)


def _k1a_call(z, W1):
    def body(z_ref, w_ref, h_ref):
        h_ref[:, :] = jnp.dot(
            z_ref[:, :], w_ref[:, :], preferred_element_type=jnp.float32
        )

    return pl.pallas_call(
        body,
        grid=(NB,),
        in_specs=[
            pl.BlockSpec((BR, IN_CH), lambda i: (i, 0)),
            pl.BlockSpec((IN_CH, HID), lambda i: (0, 0)),
        ],
        out_specs=pl.BlockSpec((BR, HID), lambda i: (i, 0)),
        out_shape=jax.ShapeDtypeStruct((N_NODES, HID), jnp.float32),
    )(z, W1)


def _k1b_call(h1, d0, d1):
    def body(h_ref, d0_ref, d1_ref, hs_ref, dinv_ref):
        deg = 1.0 + d0_ref[:, :] + d1_ref[:, :]
        dinv = lax.rsqrt(deg)
        hs_ref[:, :] = h_ref[:, :] * dinv
        dinv_ref[:, :] = dinv

    return pl.pallas_call(
        body,
        grid=(NB,),
        in_specs=[
            pl.BlockSpec((BR, HID), lambda i: (i, 0)),
            pl.BlockSpec((BR, 1), lambda i: (i, 0)),
            pl.BlockSpec((BR, 1), lambda i: (i, 0)),
        ],
        out_specs=[
            pl.BlockSpec((BR, HID), lambda i: (i, 0)),
            pl.BlockSpec((BR, 1), lambda i: (i, 0)),
        ],
        out_shape=[
            jax.ShapeDtypeStruct((N_NODES, HID), jnp.float32),
            jax.ShapeDtypeStruct((N_NODES, 1), jnp.float32),
        ],
    )(h1, d0, d1)


def _k2_call(accp1, hs1, dinv, W2, b1):
    def body(p0_ref, p1_ref, hs_ref, dinv_ref, w_ref, b_ref, o_ref):
        dinv = dinv_ref[:, :]
        t = (p0_ref[:, :] + p1_ref[:, :] + hs_ref[:, :]) * dinv + b_ref[:, :]
        r = jnp.maximum(t, 0.0)
        o_ref[:, :] = jnp.dot(r, w_ref[:, :], preferred_element_type=jnp.float32) * dinv

    return pl.pallas_call(
        body,
        grid=(NB,),
        in_specs=[
            pl.BlockSpec((BR, HID), lambda i: (i, 0)),
            pl.BlockSpec((BR, HID), lambda i: (i + NB, 0)),
            pl.BlockSpec((BR, HID), lambda i: (i, 0)),
            pl.BlockSpec((BR, 1), lambda i: (i, 0)),
            pl.BlockSpec((HID, OUT), lambda i: (0, 0)),
            pl.BlockSpec((1, HID), lambda i: (0, 0)),
        ],
        out_specs=pl.BlockSpec((BR, OUT), lambda i: (i, 0)),
        out_shape=jax.ShapeDtypeStruct((N_NODES, OUT), jnp.float32),
    )(accp1, accp1, hs1, dinv, W2, b1)


def _k3_call(accp2, hs2, dinv, Wfc, b2, bfc):
    def body(q0_ref, q1_ref, hs_ref, dinv_ref, w_ref, b2_ref, bfc_ref, o_ref):
        acc = q0_ref[:, :] + q1_ref[:, :] + hs_ref[:, :]
        t = acc * dinv_ref[:, :] + b2_ref[:, :]
        o_ref[:, :] = (
            jnp.dot(t, w_ref[:, :], preferred_element_type=jnp.float32) + bfc_ref[:, :]
        )

    return pl.pallas_call(
        body,
        grid=(NB,),
        in_specs=[
            pl.BlockSpec((BR, OUT), lambda i: (i, 0)),
            pl.BlockSpec((BR, OUT), lambda i: (i + NB, 0)),
            pl.BlockSpec((BR, OUT), lambda i: (i, 0)),
            pl.BlockSpec((BR, 1), lambda i: (i, 0)),
            pl.BlockSpec((OUT, FC), lambda i: (0, 0)),
            pl.BlockSpec((1, OUT), lambda i: (0, 0)),
            pl.BlockSpec((1, FC), lambda i: (0, 0)),
        ],
        out_specs=pl.BlockSpec((BR, FC), lambda i: (i, 0)),
        out_shape=jax.ShapeDtypeStruct((N_NODES, FC), jnp.float32),
    )(accp2, accp2, hs2, dinv, Wfc, b2, bfc)


def kernel(z, edge_index, W1, b1, W2, b2, Wfc, bfc):
    ei = edge_index.astype(jnp.int32)
    src = ei[0]
    dst = ei[1]

    degp = _deg_call(dst)
    h1 = _k1a_call(z, W1)
    d0 = degp[:N_NODES].reshape(N_NODES, 1)
    d1 = degp[N_NODES:].reshape(N_NODES, 1)
    hs1, dinv = _k1b_call(h1, d0, d1)
    accp1 = _edge_scatter_call(hs1, src, dst, HID)
    hs2 = _k2_call(accp1, hs1, dinv, W2, b1.reshape(1, HID))
    accp2 = _edge_scatter_call(hs2, src, dst, OUT, tiled=False, stage=True)
    out = _k3_call(accp2, hs2, dinv, Wfc, b2.reshape(1, OUT), bfc.reshape(1, FC))
    return out
